# baseline (speedup 1.0000x reference)
import functools

import jax
import jax.numpy as jnp
from jax import lax
from jax.experimental import pallas as pl
from jax.experimental.pallas import tpu as pltpu
from jax.experimental.pallas import tpu_sc as plsc

_NUM_CORES = 2
_NUM_SUBCORES = 16
_NUM_WORKERS = _NUM_CORES * _NUM_SUBCORES


def _gather_kernel(n_per_worker, chunk, table_hbm, idx_hbm, out_hbm,
                   idx_v, rows_v, sem):
    wid = lax.axis_index("s") * _NUM_CORES + lax.axis_index("c")
    base = wid * n_per_worker
    num_chunks = n_per_worker // chunk

    @pl.loop(0, num_chunks)
    def _chunk_loop(i):
        off = base + i * chunk
        pltpu.sync_copy(idx_hbm.at[pl.ds(off, chunk)], idx_v)
        pltpu.async_copy(table_hbm.at[idx_v], rows_v, sem).wait()
        pltpu.sync_copy(rows_v, out_hbm.at[pl.ds(off, chunk)])


@jax.jit
def kernel(x, table):
    batch, seq = x.shape
    vocab, dim = table.shape
    n = batch * seq
    assert n % _NUM_WORKERS == 0
    n_per_worker = n // _NUM_WORKERS
    chunk = 1600
    assert n_per_worker % chunk == 0

    idx = x.reshape(n).astype(jnp.int32)

    mesh = plsc.VectorSubcoreMesh(
        core_axis_name="c", subcore_axis_name="s",
        num_cores=_NUM_CORES, num_subcores=_NUM_SUBCORES)

    out = pl.kernel(
        functools.partial(_gather_kernel, n_per_worker, chunk),
        out_type=jax.ShapeDtypeStruct((n, dim), jnp.float32),
        mesh=mesh,
        scratch_types=[
            pltpu.VMEM((chunk,), jnp.int32),
            pltpu.VMEM((chunk, dim), jnp.float32),
            pltpu.SemaphoreType.DMA,
        ],
        compiler_params=pltpu.CompilerParams(use_tc_tiling_on_sc=False),
    )(table, idx)

    return out.reshape(batch, seq, dim)


# speedup vs baseline: 1.0000x; 1.0000x over previous
import functools

import jax
import jax.numpy as jnp
from jax import lax
from jax.experimental import pallas as pl
from jax.experimental.pallas import tpu as pltpu
from jax.experimental.pallas import tpu_sc as plsc

_NUM_CORES = 2
_NUM_SUBCORES = 16
_NUM_WORKERS = _NUM_CORES * _NUM_SUBCORES


def _gather_kernel(n_per_worker, chunk, table_hbm, idx_hbm, out_hbm,
                   idx_v, rows_v, sem_g0, sem_g1, sem_s0, sem_s1):
    wid = lax.axis_index("s") * _NUM_CORES + lax.axis_index("c")
    base = wid * n_per_worker
    num_chunks = n_per_worker // chunk
    gather_sems = (sem_g0, sem_g1)
    store_sems = (sem_s0, sem_s1)

    pltpu.sync_copy(idx_hbm.at[pl.ds(base, n_per_worker)], idx_v)

    store_copies = [None, None]
    prev = None
    for i in range(num_chunks):
        b = i & 1
        if store_copies[b] is not None:
            store_copies[b].wait()
        g = pltpu.async_copy(
            table_hbm.at[idx_v.at[pl.ds(i * chunk, chunk)]],
            rows_v.at[b], gather_sems[b])
        if prev is not None:
            pg, pb, poff = prev
            pg.wait()
            store_copies[pb] = pltpu.async_copy(
                rows_v.at[pb], out_hbm.at[pl.ds(poff, chunk)], store_sems[pb])
        prev = (g, b, base + i * chunk)
    pg, pb, poff = prev
    pg.wait()
    pltpu.sync_copy(rows_v.at[pb], out_hbm.at[pl.ds(poff, chunk)])
    if store_copies[1 - pb] is not None:
        store_copies[1 - pb].wait()


@jax.jit
def kernel(x, table):
    batch, seq = x.shape
    vocab, dim = table.shape
    n = batch * seq
    assert n % _NUM_WORKERS == 0
    n_per_worker = n // _NUM_WORKERS
    chunk = 800
    assert n_per_worker % chunk == 0

    idx = x.reshape(n).astype(jnp.int32)

    mesh = plsc.VectorSubcoreMesh(
        core_axis_name="c", subcore_axis_name="s",
        num_cores=_NUM_CORES, num_subcores=_NUM_SUBCORES)

    out = pl.kernel(
        functools.partial(_gather_kernel, n_per_worker, chunk),
        out_type=jax.ShapeDtypeStruct((n, dim), jnp.float32),
        mesh=mesh,
        scratch_types=[
            pltpu.VMEM((n_per_worker,), jnp.int32),
            pltpu.VMEM((2, chunk, dim), jnp.float32),
            pltpu.SemaphoreType.DMA,
            pltpu.SemaphoreType.DMA,
            pltpu.SemaphoreType.DMA,
            pltpu.SemaphoreType.DMA,
        ],
        compiler_params=pltpu.CompilerParams(use_tc_tiling_on_sc=False),
    )(table, idx)

    return out.reshape(batch, seq, dim)
